# Initial kernel scaffold; baseline (speedup 1.0000x reference)
import jax
import jax.numpy as jnp
from jax.experimental import pallas as pl


def kernel(inputs, embedding):
    raise NotImplementedError("write your pallas kernel here")


# speedup vs baseline: 1.2817x; 1.2817x over previous
import jax
import jax.numpy as jnp
from jax import lax
from jax.experimental import pallas as pl
from jax.experimental.pallas import tpu as pltpu
from jax.experimental.pallas import tpu_sc as plsc

_info = plsc.get_sparse_core_info()
_NC, _NS = _info.num_cores, _info.num_subcores
_NW = _NC * _NS

_ROWS = 4096 * 26
_D = 128
_CHUNK = 128
_PER_W = _ROWS // _NW
_NCHUNK = _PER_W // _CHUNK
_NBUF = 2


def _sc_gather(table, idx3):
  mesh = plsc.VectorSubcoreMesh(core_axis_name="c", subcore_axis_name="s")

  def body(table_hbm, idx_hbm, out_hbm, idx_v, rows_v, sem0, sem1):
    sems = (sem0, sem1)
    wid = lax.axis_index("s") * _NC + lax.axis_index("c")
    base = wid * _PER_W
    pltpu.sync_copy(idx_hbm.at[wid], idx_v)
    for b in range(_NBUF):
      pltpu.async_copy(table_hbm.at[idx_v.at[b]], rows_v.at[b], sems[b])

    @pl.loop(0, _NCHUNK, step=_NBUF)
    def _(c0):
      for b in range(_NBUF):
        c = c0 + b
        pltpu.make_async_copy(
            table_hbm.at[idx_v.at[c]], rows_v.at[b], sems[b]).wait()
        pltpu.sync_copy(
            rows_v.at[b], out_hbm.at[pl.ds(base + c * _CHUNK, _CHUNK)])
        nxt = c + _NBUF

        @pl.when(nxt < _NCHUNK)
        def _():
          pltpu.async_copy(table_hbm.at[idx_v.at[nxt]], rows_v.at[b], sems[b])

  f = pl.kernel(
      body,
      out_type=jax.ShapeDtypeStruct((_ROWS, _D), jnp.float32),
      mesh=mesh,
      scratch_types=[
          pltpu.VMEM((_NCHUNK, _CHUNK), jnp.int32),
          pltpu.VMEM((_NBUF, _CHUNK, _D), jnp.float32),
          pltpu.SemaphoreType.DMA,
          pltpu.SemaphoreType.DMA,
      ],
  )
  return f(table, idx3)


def kernel(inputs, embedding):
  idx3 = inputs.astype(jnp.int32).reshape(_NW, _NCHUNK, _CHUNK)
  out = _sc_gather(embedding, idx3)
  return out.reshape(inputs.shape[0], inputs.shape[1], _D)


# speedup vs baseline: 1.2961x; 1.0112x over previous
import jax
import jax.numpy as jnp
from jax import lax
from jax.experimental import pallas as pl
from jax.experimental.pallas import tpu as pltpu
from jax.experimental.pallas import tpu_sc as plsc

_info = plsc.get_sparse_core_info()
_NC, _NS = _info.num_cores, _info.num_subcores
_NW = _NC * _NS

_ROWS = 4096 * 26
_D = 128
_CHUNK = 416
_PER_W = _ROWS // _NW
_NCHUNK = _PER_W // _CHUNK
_NBUF = 2


def _sc_gather(table, idx3):
  mesh = plsc.VectorSubcoreMesh(core_axis_name="c", subcore_axis_name="s")

  def body(table_hbm, idx_hbm, out_hbm, idx_v, rows_v, sem0, sem1):
    sems = (sem0, sem1)
    wid = lax.axis_index("s") * _NC + lax.axis_index("c")
    base = wid * _PER_W
    pltpu.sync_copy(idx_hbm.at[wid], idx_v)
    for b in range(_NBUF):
      pltpu.async_copy(
          table_hbm.at[idx_v.at[pl.ds(b * _CHUNK, _CHUNK)]],
          rows_v.at[b], sems[b])

    @pl.loop(0, _NCHUNK, step=_NBUF)
    def _(c0):
      for b in range(_NBUF):
        c = c0 + b
        pltpu.make_async_copy(
            table_hbm.at[idx_v.at[pl.ds(c * _CHUNK, _CHUNK)]],
            rows_v.at[b], sems[b]).wait()
        pltpu.sync_copy(
            rows_v.at[b], out_hbm.at[pl.ds(base + c * _CHUNK, _CHUNK)])
        nxt = c + _NBUF

        @pl.when(nxt < _NCHUNK)
        def _():
          pltpu.async_copy(
              table_hbm.at[idx_v.at[pl.ds(nxt * _CHUNK, _CHUNK)]],
              rows_v.at[b], sems[b])

  f = pl.kernel(
      body,
      out_type=jax.ShapeDtypeStruct((_ROWS, _D), jnp.float32),
      mesh=mesh,
      scratch_types=[
          pltpu.VMEM((_PER_W,), jnp.int32),
          pltpu.VMEM((_NBUF, _CHUNK, _D), jnp.float32),
          pltpu.SemaphoreType.DMA,
          pltpu.SemaphoreType.DMA,
      ],
  )
  return f(table, idx3)


def kernel(inputs, embedding):
  idx3 = inputs.astype(jnp.int32).reshape(_NW, _PER_W)
  out = _sc_gather(embedding, idx3)
  return out.reshape(inputs.shape[0], inputs.shape[1], _D)


# speedup vs baseline: 1.3000x; 1.0030x over previous
import jax
import jax.numpy as jnp
from jax import lax
from jax.experimental import pallas as pl
from jax.experimental.pallas import tpu as pltpu
from jax.experimental.pallas import tpu_sc as plsc

_info = plsc.get_sparse_core_info()
_NC, _NS = _info.num_cores, _info.num_subcores
_NW = _NC * _NS

_ROWS = 4096 * 26
_D = 128
_CHUNK = 208
_PER_W = _ROWS // _NW
_NCHUNK = _PER_W // _CHUNK
_NBUF = 4
_LAG = 3


def _sc_gather(table, idx2):
  mesh = plsc.VectorSubcoreMesh(core_axis_name="c", subcore_axis_name="s")

  def body(table_hbm, idx_hbm, out_hbm, idx_v, rows_v, gsems, ssems):
    wid = lax.axis_index("s") * _NC + lax.axis_index("c")
    base = wid * _PER_W
    pltpu.sync_copy(idx_hbm.at[wid], idx_v)

    def gather(c, b):
      return pltpu.make_async_copy(
          table_hbm.at[idx_v.at[pl.ds(c * _CHUNK, _CHUNK)]],
          rows_v.at[b], gsems.at[b])

    def store(c, b):
      return pltpu.make_async_copy(
          rows_v.at[b], out_hbm.at[pl.ds(base + c * _CHUNK, _CHUNK)],
          ssems.at[b])

    for c in range(_NCHUNK + _LAG):
      b = c % _NBUF
      if c < _NCHUNK:
        if c >= _NBUF:
          store(c - _NBUF, b).wait()
        gather(c, b).start()
      d = c - _LAG
      if d >= 0:
        bd = d % _NBUF
        gather(d, bd).wait()
        store(d, bd).start()
    for d in range(_NCHUNK - _NBUF, _NCHUNK):
      store(d, d % _NBUF).wait()

  f = pl.kernel(
      body,
      out_type=jax.ShapeDtypeStruct((_ROWS, _D), jnp.float32),
      mesh=mesh,
      scratch_types=[
          pltpu.VMEM((_PER_W,), jnp.int32),
          pltpu.VMEM((_NBUF, _CHUNK, _D), jnp.float32),
          pltpu.SemaphoreType.DMA((_NBUF,)),
          pltpu.SemaphoreType.DMA((_NBUF,)),
      ],
  )
  return f(table, idx2)


def kernel(inputs, embedding):
  idx2 = inputs.astype(jnp.int32).reshape(_NW, _PER_W)
  out = _sc_gather(embedding, idx2)
  return out.reshape(inputs.shape[0], inputs.shape[1], _D)
